# baseline (speedup 1.0000x reference)
import functools

import jax
import jax.numpy as jnp
from jax import lax
from jax.experimental import pallas as pl
from jax.experimental.pallas import tpu as pltpu
from jax.experimental.pallas import tpu_sc as plsc

_HIDDEN = 64
_MAX_LEN = 2048
_TABLE_ROWS = 2 * _MAX_LEN - 1
_SPLIT = 1024


def _make_sc_kernel(S: int, D: int, T: int, row_lo: int, n_rows: int):
    info = plsc.get_sparse_core_info()
    num_cores, num_subcores = info.num_cores, info.num_subcores
    num_workers = num_cores * num_subcores
    rows_per_worker = n_rows // num_workers
    depth = 8

    mesh = plsc.VectorSubcoreMesh(core_axis_name="c", subcore_axis_name="s")

    @functools.partial(
        pl.kernel,
        mesh=mesh,
        out_type=jax.ShapeDtypeStruct((n_rows, S, D), jnp.float32),
        scratch_types=[
            pltpu.VMEM_SHARED((T, D), jnp.float32),
            pltpu.SemaphoreType.DMA,
        ],
    )
    def sc_kernel(table_hbm, out_hbm, spmem_table, sem):
        c = lax.axis_index("c")
        s = lax.axis_index("s")

        @pl.when(s == 0)
        def _stage():
            pltpu.sync_copy(table_hbm, spmem_table)

        plsc.subcore_barrier()

        wid = c * num_subcores + s
        base = wid * rows_per_worker

        def copy_descr(local_i):
            i = row_lo + local_i
            src = spmem_table.at[pl.ds((S - 1) - i, S)]
            return pltpu.make_async_copy(src, out_hbm.at[local_i], sem)

        for t in range(depth):
            copy_descr(base + t).start()

        @pl.loop(0, rows_per_worker - depth)
        def _steady(k):
            copy_descr(base + k).wait()
            copy_descr(base + k + depth).start()

        @pl.loop(0, depth)
        def _drain(k):
            copy_descr(base + rows_per_worker - depth + k).wait()

    return sc_kernel


def _make_tc_kernel(S: int, D: int, n_rows: int, block_rows: int = 8):
    T_pad = 2 * S

    def body(table_ref, out_ref):
        b = pl.program_id(0)
        for ii in range(block_rows):
            i = b * block_rows + ii
            out_ref[ii] = table_ref[pl.ds((S - 1) - i, S), :]

    return pl.pallas_call(
        body,
        grid=(n_rows // block_rows,),
        in_specs=[pl.BlockSpec((T_pad, D), lambda b: (0, 0))],
        out_specs=pl.BlockSpec((block_rows, S, D), lambda b: (b, 0, 0)),
        out_shape=jax.ShapeDtypeStruct((n_rows, S, D), jnp.float32),
    )


_sc_kernel = _make_sc_kernel(_MAX_LEN, _HIDDEN, _TABLE_ROWS, _SPLIT,
                             _MAX_LEN - _SPLIT)
_tc_kernel = _make_tc_kernel(_MAX_LEN, _HIDDEN, _SPLIT)


def kernel(x, positional_params):
    del x
    table_pad = jnp.pad(positional_params, ((0, 1), (0, 0)))
    sc_part = _sc_kernel(positional_params)
    tc_part = _tc_kernel(table_pad)
    return jnp.concatenate([tc_part, sc_part], axis=0)
